# baseline (speedup 1.0000x reference)
import functools

import jax
import jax.numpy as jnp
from jax import lax
from jax.experimental import pallas as pl
from jax.experimental.pallas import tpu as pltpu
from jax.experimental.pallas import tpu_sc as plsc

N_USERS = 5000
N_ITEMS = 5000
N = N_USERS + N_ITEMS
D = 16
EPS = 1e-12

BM = 400
NM = N // BM


def _gcn_body(a_ref, e0_ref, out_ref, pad_ref, c1_ref):
    l = pl.program_id(0)
    m = pl.program_id(1)
    row = pl.ds(m * BM, BM)
    cur = jnp.where(l == 0, e0_ref[...], c1_ref[...])
    acc = jnp.dot(a_ref[...], cur, preferred_element_type=jnp.float32)
    nrm = jnp.sqrt(jnp.sum(acc * acc, axis=1, keepdims=True))
    c = acc / jnp.maximum(nrm, EPS)

    @pl.when(l == 0)
    def _():
        c1_ref[row, :] = c
        out_ref[...] = c

    @pl.when(l == 1)
    def _():
        mean = (e0_ref[row, :] + c1_ref[row, :] + c) * (1.0 / 3.0)
        out_ref[...] = mean
        pad_ref[:, 0:D] = mean


def _propagate(a, e0, interpret=False):
    return pl.pallas_call(
        _gcn_body,
        grid=(2, NM),
        in_specs=[
            pl.BlockSpec((BM, N), lambda l, m: (m, 0)),
            pl.BlockSpec((N, D), lambda l, m: (0, 0)),
        ],
        out_specs=[
            pl.BlockSpec((BM, D), lambda l, m: (m, 0)),
            pl.BlockSpec((BM, 128), lambda l, m: (m, 0)),
        ],
        out_shape=[
            jax.ShapeDtypeStruct((N, D), jnp.float32),
            jax.ShapeDtypeStruct((N, 128), jnp.float32),
        ],
        scratch_shapes=[pltpu.VMEM((N, D), jnp.float32)],
        interpret=interpret,
    )(a, e0)


B = 4096
NW = 32
BPW = B // NW

_SC_MESH = plsc.VectorSubcoreMesh(core_axis_name="c", subcore_axis_name="s")


@functools.partial(
    pl.kernel,
    mesh=_SC_MESH,
    out_type=jax.ShapeDtypeStruct((3 * B, 128), jnp.float32),
    scratch_types=[
        pltpu.VMEM((3 * BPW,), jnp.int32),
        pltpu.VMEM((3 * BPW, 128), jnp.float32),
        pltpu.VMEM((3 * BPW,), jnp.int32),
        pltpu.SemaphoreType.DMA,
        pltpu.SemaphoreType.DMA,
    ],
)
def _sc_gather(emb_hbm, uid_hbm, pid_hbm, nid_hbm, out_hbm,
               idx, rows, _unused, sem, wsem):
    wid = lax.axis_index("s") * 2 + lax.axis_index("c")
    base = wid * BPW
    iu = pltpu.async_copy(uid_hbm.at[pl.ds(base, BPW)], idx.at[pl.ds(0, BPW)], wsem)
    ip = pltpu.async_copy(pid_hbm.at[pl.ds(base, BPW)], idx.at[pl.ds(BPW, BPW)], wsem)
    inn = pltpu.async_copy(nid_hbm.at[pl.ds(base, BPW)], idx.at[pl.ds(2 * BPW, BPW)], wsem)
    iu.wait()
    ip.wait()
    inn.wait()
    pltpu.async_copy(emb_hbm.at[idx], rows, sem).wait()
    pltpu.sync_copy(rows, out_hbm.at[pl.ds(wid * 3 * BPW, 3 * BPW)])


def _loss_body(g_ref, out_ref, acc_ref):
    w = pl.program_id(0)
    u = g_ref[0:BPW, 0:D]
    diff = (jnp.sum(u * g_ref[BPW:2 * BPW, 0:D], axis=1)
            - jnp.sum(u * g_ref[2 * BPW:3 * BPW, 0:D], axis=1))
    part = jnp.sum(jnp.log(jax.nn.sigmoid(diff)))

    @pl.when(w == 0)
    def _():
        acc_ref[0] = part

    @pl.when(w > 0)
    def _():
        acc_ref[0] += part

    @pl.when(w == NW - 1)
    def _():
        out_ref[0, 0] = -acc_ref[0] * (1.0 / B)


def _loss(rows):
    out = pl.pallas_call(
        _loss_body,
        grid=(NW,),
        in_specs=[pl.BlockSpec((3 * BPW, 128), lambda w: (w, 0))],
        out_specs=pl.BlockSpec(memory_space=pltpu.SMEM),
        out_shape=jax.ShapeDtypeStruct((1, 1), jnp.float32),
        scratch_shapes=[pltpu.SMEM((1,), jnp.float32)],
    )(rows)
    return out[0, 0]


def kernel(user_emb, item_emb, edge_index, user_id, pos_item, neg_item):
    e0 = jnp.concatenate([user_emb, item_emb], axis=0)
    all_emb, pad = _propagate(edge_index, e0)
    rows = _sc_gather(pad, user_id, pos_item + N_USERS, neg_item + N_USERS)
    rec_loss = _loss(rows)
    return (rec_loss, all_emb)


# speedup vs baseline: 1.0528x; 1.0528x over previous
import functools

import jax
import jax.numpy as jnp
from jax import lax
from jax.experimental import pallas as pl
from jax.experimental.pallas import tpu as pltpu
from jax.experimental.pallas import tpu_sc as plsc

N_USERS = 5000
N_ITEMS = 5000
N = N_USERS + N_ITEMS
D = 16
EPS = 1e-12

BM = 400
NM = N // BM


def _gcn_body(a_ref, e0_ref, out_ref, pad_ref, c1_ref):
    l = pl.program_id(0)
    m = pl.program_id(1)
    row = pl.ds(m * BM, BM)
    cur = jnp.where(l == 0, e0_ref[...], c1_ref[...])
    acc = jnp.dot(a_ref[...], cur, preferred_element_type=jnp.float32)
    nrm = jnp.sqrt(jnp.sum(acc * acc, axis=1, keepdims=True))
    c = acc / jnp.maximum(nrm, EPS)

    @pl.when(l == 0)
    def _():
        c1_ref[row, :] = c
        out_ref[...] = c

    @pl.when(l == 1)
    def _():
        mean = (e0_ref[row, :] + c1_ref[row, :] + c) * (1.0 / 3.0)
        out_ref[...] = mean
        pad_ref[:, 0:D] = mean


def _propagate(a, e0, interpret=False):
    return pl.pallas_call(
        _gcn_body,
        grid=(2, NM),
        in_specs=[
            pl.BlockSpec((BM, N), lambda l, m: (m, 0)),
            pl.BlockSpec((N, D), lambda l, m: (0, 0)),
        ],
        out_specs=[
            pl.BlockSpec((BM, D), lambda l, m: (m, 0)),
            pl.BlockSpec((BM, 128), lambda l, m: (m, 0)),
        ],
        out_shape=[
            jax.ShapeDtypeStruct((N, D), jnp.float32),
            jax.ShapeDtypeStruct((N, 128), jnp.float32),
        ],
        scratch_shapes=[pltpu.VMEM((N, D), jnp.float32)],
        interpret=interpret,
    )(a, e0)


B = 4096
NW = 32
BPW = B // NW

_SC_MESH = plsc.VectorSubcoreMesh(core_axis_name="c", subcore_axis_name="s")


@functools.partial(
    pl.kernel,
    mesh=_SC_MESH,
    out_type=jax.ShapeDtypeStruct((3 * B, 128), jnp.float32),
    scratch_types=[
        pltpu.VMEM((BPW,), jnp.int32),
        pltpu.VMEM((BPW,), jnp.int32),
        pltpu.VMEM((BPW,), jnp.int32),
        pltpu.VMEM((BPW, 128), jnp.float32),
        pltpu.VMEM((BPW, 128), jnp.float32),
        pltpu.VMEM((BPW, 128), jnp.float32),
        pltpu.SemaphoreType.DMA,
        pltpu.SemaphoreType.DMA,
    ],
)
def _sc_gather(emb_hbm, uid_hbm, pid_hbm, nid_hbm, out_hbm,
               uidx, pidx, nidx, urows, prows, nrows, sem, wsem):
    wid = lax.axis_index("s") * 2 + lax.axis_index("c")
    base = wid * BPW
    iu = pltpu.async_copy(uid_hbm.at[pl.ds(base, BPW)], uidx, wsem)
    ip = pltpu.async_copy(pid_hbm.at[pl.ds(base, BPW)], pidx, wsem)
    inn = pltpu.async_copy(nid_hbm.at[pl.ds(base, BPW)], nidx, wsem)
    iu.wait()
    cu = pltpu.async_copy(emb_hbm.at[uidx], urows, sem)
    ip.wait()
    cp = pltpu.async_copy(emb_hbm.at[pidx], prows, sem)
    inn.wait()
    cn = pltpu.async_copy(emb_hbm.at[nidx], nrows, sem)
    cu.wait()
    wu = pltpu.async_copy(urows, out_hbm.at[pl.ds(base, BPW)], wsem)
    cp.wait()
    wp = pltpu.async_copy(prows, out_hbm.at[pl.ds(B + base, BPW)], wsem)
    cn.wait()
    wn = pltpu.async_copy(nrows, out_hbm.at[pl.ds(2 * B + base, BPW)], wsem)
    wu.wait()
    wp.wait()
    wn.wait()


NC = 4
CB = B // NC


def _loss_body(u_ref, p_ref, n_ref, out_ref, acc_ref):
    i = pl.program_id(0)
    u = u_ref[:, 0:D]
    diff = (jnp.sum(u * p_ref[:, 0:D], axis=1)
            - jnp.sum(u * n_ref[:, 0:D], axis=1))
    part = jnp.sum(jnp.log(jax.nn.sigmoid(diff)))

    @pl.when(i == 0)
    def _():
        acc_ref[0] = part

    @pl.when(i > 0)
    def _():
        acc_ref[0] += part

    @pl.when(i == NC - 1)
    def _():
        out_ref[0, 0] = -acc_ref[0] * (1.0 / B)


def _loss(rows):
    out = pl.pallas_call(
        _loss_body,
        grid=(NC,),
        in_specs=[
            pl.BlockSpec((CB, 128), lambda i: (i, 0)),
            pl.BlockSpec((CB, 128), lambda i: (i + NC, 0)),
            pl.BlockSpec((CB, 128), lambda i: (i + 2 * NC, 0)),
        ],
        out_specs=pl.BlockSpec(memory_space=pltpu.SMEM),
        out_shape=jax.ShapeDtypeStruct((1, 1), jnp.float32),
        scratch_shapes=[pltpu.SMEM((1,), jnp.float32)],
    )(rows, rows, rows)
    return out[0, 0]


def kernel(user_emb, item_emb, edge_index, user_id, pos_item, neg_item):
    e0 = jnp.concatenate([user_emb, item_emb], axis=0)
    all_emb, pad = _propagate(edge_index, e0)
    rows = _sc_gather(pad, user_id, pos_item + N_USERS, neg_item + N_USERS)
    rec_loss = _loss(rows)
    return (rec_loss, all_emb)
